# baseline (speedup 1.0000x reference)
import functools
import jax
import jax.numpy as jnp
from jax import lax
from jax.experimental import pallas as pl
from jax.experimental.pallas import tpu as pltpu
from jax.experimental.pallas import tpu_sc as plsc

N = 10000
E = 320000
D = 128
R = 8

NS = 16
NHALF = 5120
EPT = 20480
EPAD = NS * EPT
GB = 64
NSH = 5760
STRIDE = NSH // NS
NPEL = 10256
NB_TC = 10
BN = N // NB_TC
BC = 80
NBC = N // BC
EROWS = 2560


def _tc_pre_body(x_ref, w_ref, al_ref, ar_ref, src_ref, dst_ref, rt_ref,
                 xr_ref, s2_ref, el_ref, pk_ref):
    r = pl.program_id(1)
    xb = x_ref[...]
    w = w_ref[0]
    xr = jnp.dot(xb, w, preferred_element_type=jnp.float32)
    xr_ref[...] = xr
    s2_ref[...] = jnp.dot(xr, ar_ref[...]).reshape(BN, 1)

    @pl.when(r == 0)
    def _():
        el_ref[...] = jnp.dot(xb, al_ref[...]).reshape(BN, 1)
        key = rt_ref[...] * N + src_ref[...]
        pk_ref[...] = key * 16384 + dst_ref[...]


def _phase_a(x, Wrel, attn_l, attn_r, src2, dst2, rt2):
    eblk = pl.BlockSpec((EROWS // NB_TC, 128), lambda n, r: (n, 0))
    return pl.pallas_call(
        _tc_pre_body,
        grid=(NB_TC, R),
        in_specs=[
            pl.BlockSpec((BN, D), lambda n, r: (n, 0)),
            pl.BlockSpec((1, D, D), lambda n, r: (r, 0, 0)),
            pl.BlockSpec((D,), lambda n, r: (0,)),
            pl.BlockSpec((D,), lambda n, r: (0,)),
            eblk, eblk, eblk,
        ],
        out_specs=[
            pl.BlockSpec((BN, D), lambda n, r: (r * NB_TC + n, 0)),
            pl.BlockSpec((BN, 1), lambda n, r: (r * NB_TC + n, 0)),
            pl.BlockSpec((BN, 1), lambda n, r: (n, 0)),
            eblk,
        ],
        out_shape=[
            jax.ShapeDtypeStruct((R * N, D), jnp.float32),
            jax.ShapeDtypeStruct((R * N, 1), jnp.float32),
            jax.ShapeDtypeStruct((N, 1), jnp.float32),
            jax.ShapeDtypeStruct((EROWS, 128), jnp.int32),
        ],
    )(x, Wrel, attn_l, attn_r, src2, dst2, rt2)


def _sc_body(xr_h, s2_h, el_h, pk_h, U_h, den_h,
             el_t, pk_t, mpk, rows, srows, kb0, kb1, lb0, lb1, pb0, pb1,
             s2b0, s2b1, zbuf,
             semr0, semr1, sems0, sems1, semu0, semu1, semd0, semd1,
             U_sh, den_sh):
    c = lax.axis_index("c")
    sid = lax.axis_index("s")
    nlo = c * NHALF
    eb = sid * EPT

    zf = jnp.zeros((16,), jnp.float32)
    zi = jnp.zeros((16,), jnp.int32)
    iota16 = lax.iota(jnp.int32, 16)
    m14 = zi + 16383
    s14 = zi + 14
    vH = zi + NHALF
    vE = zi + E
    vT = zi + NHALF
    f02 = zf + 0.2

    pltpu.sync_copy(el_h.at[pl.ds(nlo, NPEL // 2)], el_t)
    pltpu.sync_copy(pk_h.at[pl.ds(eb, EPT)], pk_t)

    @pl.loop(0, 64, unroll=8)
    def _zs(i):
        for j in range(8):
            srows[i, pl.ds(j * 16, 16)] = zf

    @pl.loop(0, STRIDE // 16, unroll=8)
    def _zb(i):
        zbuf[pl.ds(i * 16, 16)] = zf

    pltpu.sync_copy(zbuf, den_sh.at[pl.ds(sid * STRIDE, STRIDE)])
    for k in range(5):
        pltpu.sync_copy(srows.at[pl.ds(0, 64)],
                        U_sh.at[pl.ds(sid * STRIDE + k * 64, 64)])
    pltpu.sync_copy(srows.at[pl.ds(0, 40)],
                    U_sh.at[pl.ds(sid * STRIDE + 320, 40)])
    plsc.subcore_barrier()

    def fbody(i, cnt):
        pk = pk_t[pl.ds(i * 16, 16)]
        ld = (pk & m14) - nlo
        gv = (zi + (eb + i * 16)) + iota16
        msk = (ld >= zi) & (ld < vH) & (gv < vE)
        plsc.store_compressed(mpk.at[pl.ds(cnt, 16)], pk, mask=msk)
        pc = plsc.all_reduce_population_count(msk)[0]
        return cnt + pc

    m = lax.fori_loop(0, EPT // 16, fbody, jnp.int32(0))
    nb = (m + GB - 1) // GB
    mv = jnp.broadcast_to(m, (16,))

    @pl.loop(0, 9)
    def _ztail(i):
        mpk[pl.ds(m + i * 16, 16)] = zi

    def prep_issue(b, kb, rb, semr, sems, s2b):
        boff = b * GB
        for t in range(GB // 16):
            pk = mpk[pl.ds(boff + t * 16, 16)]
            kb[pl.ds(t * 16, 16)] = lax.shift_right_logical(pk, s14)
        pltpu.async_copy(xr_h.at[kb], rows.at[pl.ds(rb, GB)], semr)
        pltpu.async_copy(s2_h.at[kb], s2b, sems)

    def process(b, rb, kb, lb, pb, s2b, semr, sems, semu, semd):
        @pl.when(b >= 2)
        def _():
            pltpu.make_async_copy(srows.at[pl.ds(rb, GB)], U_sh.at[lb],
                                  semu).wait()
            pltpu.make_async_copy(pb.at[pl.ds(0, GB)], den_sh.at[lb],
                                  semd).wait()
        boff = b * GB
        pltpu.make_async_copy(s2_h.at[kb], s2b, sems).wait()
        for t in range(GB // 16):
            pk = mpk[pl.ds(boff + t * 16, 16)]
            ld = (pk & m14) - nlo
            gv = (zi + (boff + t * 16)) + iota16
            ldm = jnp.where(gv < mv, ld, vT)
            lb[pl.ds(t * 16, 16)] = ldm
            eld = plsc.load_gather(el_t, [ldm])
            lg = eld + s2b[pl.ds(t * 16, 16)]
            lr = jnp.where(lg >= zf, lg, lg * f02)
            pb[pl.ds(t * 16, 16)] = jnp.exp(lr)
        pltpu.make_async_copy(xr_h.at[kb], rows.at[pl.ds(rb, GB)],
                              semr).wait()

        @pl.loop(0, GB // 2, unroll=2)
        def _scale(i):
            i2 = 2 * i
            pva = jnp.broadcast_to(pb[pl.ds(i2, 16)][0], (16,))
            pvb = jnp.broadcast_to(pb[pl.ds(i2 + 1, 16)][0], (16,))
            va = [rows[rb + i2, pl.ds(j * 16, 16)] for j in range(8)]
            vb = [rows[rb + i2 + 1, pl.ds(j * 16, 16)] for j in range(8)]
            pa = [pva * v for v in va]
            pb2 = [pvb * v for v in vb]
            for j in range(8):
                srows[rb + i2, pl.ds(j * 16, 16)] = pa[j]
            for j in range(8):
                srows[rb + i2 + 1, pl.ds(j * 16, 16)] = pb2[j]

        pltpu.async_copy(srows.at[pl.ds(rb, GB)], U_sh.at[lb], semu,
                         add=True)
        pltpu.async_copy(pb.at[pl.ds(0, GB)], den_sh.at[lb], semd, add=True)

        @pl.when(b + 2 < nb)
        def _():
            prep_issue(b + 2, kb, rb, semr, sems, s2b)

    @pl.when(nb > 0)
    def _():
        prep_issue(0, kb0, 0, semr0, sems0, s2b0)

    @pl.when(nb > 1)
    def _():
        prep_issue(1, kb1, GB, semr1, sems1, s2b1)

    @pl.loop(0, (nb + 1) // 2)
    def _pairs(u):
        b0 = 2 * u
        b1 = 2 * u + 1
        process(b0, 0, kb0, lb0, pb0, s2b0, semr0, sems0, semu0, semd0)

        @pl.when(b1 < nb)
        def _():
            process(b1, GB, kb1, lb1, pb1, s2b1, semr1, sems1, semu1, semd1)

    @pl.when(nb > 0)
    def _():
        pltpu.make_async_copy(srows.at[pl.ds(0, GB)], U_sh.at[lb0],
                              semu0).wait()
        pltpu.make_async_copy(pb0.at[pl.ds(0, GB)], den_sh.at[lb0],
                              semd0).wait()

    @pl.when(nb > 1)
    def _():
        pltpu.make_async_copy(srows.at[pl.ds(GB, GB)], U_sh.at[lb1],
                              semu1).wait()
        pltpu.make_async_copy(pb1.at[pl.ds(0, GB)], den_sh.at[lb1],
                              semd1).wait()

    plsc.subcore_barrier()

    ob = c * NSH + sid * STRIDE
    pltpu.sync_copy(U_sh.at[pl.ds(sid * STRIDE, STRIDE)],
                    U_h.at[pl.ds(ob, STRIDE)])
    pltpu.sync_copy(den_sh.at[pl.ds(sid * STRIDE, STRIDE)], zbuf)
    pltpu.sync_copy(zbuf, den_h.at[pl.ds(ob, STRIDE)])


_sc_phase = functools.partial(
    pl.kernel,
    out_type=(
        jax.ShapeDtypeStruct((2 * NSH, D), jnp.float32),
        jax.ShapeDtypeStruct((2 * NSH,), jnp.float32),
    ),
    mesh=plsc.VectorSubcoreMesh(core_axis_name="c", subcore_axis_name="s"),
    compiler_params=pltpu.CompilerParams(needs_layout_passes=False,
                                        use_tc_tiling_on_sc=False),
    scratch_types=(
        pltpu.VMEM((NPEL // 2,), jnp.float32),
        pltpu.VMEM((EPT,), jnp.int32),
        pltpu.VMEM((EPT + 160,), jnp.int32),
        pltpu.VMEM((2 * GB, D), jnp.float32),
        pltpu.VMEM((2 * GB, D), jnp.float32),
        pltpu.VMEM((GB,), jnp.int32),
        pltpu.VMEM((GB,), jnp.int32),
        pltpu.VMEM((GB,), jnp.int32),
        pltpu.VMEM((GB,), jnp.int32),
        pltpu.VMEM((GB + 16,), jnp.float32),
        pltpu.VMEM((GB + 16,), jnp.float32),
        pltpu.VMEM((GB,), jnp.float32),
        pltpu.VMEM((GB,), jnp.float32),
        pltpu.VMEM((STRIDE,), jnp.float32),
        pltpu.SemaphoreType.DMA,
        pltpu.SemaphoreType.DMA,
        pltpu.SemaphoreType.DMA,
        pltpu.SemaphoreType.DMA,
        pltpu.SemaphoreType.DMA,
        pltpu.SemaphoreType.DMA,
        pltpu.SemaphoreType.DMA,
        pltpu.SemaphoreType.DMA,
        pltpu.VMEM_SHARED((NSH, D), jnp.float32),
        pltpu.VMEM_SHARED((NSH,), jnp.float32),
    ),
)(_sc_body)


def _tc_gru_body(x_ref, U_ref, den_ref, dm_ref, Wz_ref, Uz_ref, bz_ref,
                 Wr_ref, Ur_ref, br_ref, Wh_ref, Uh_ref, bh_ref, h_ref):
    xb = x_ref[...]
    red = U_ref[...] / (den_ref[...] + 1e-9)
    xm = xb * dm_ref[...]
    dot = lambda a, b: jnp.dot(a, b, preferred_element_type=jnp.float32)
    z = jax.nn.sigmoid(dot(xm, Wz_ref[...]) + dot(red, Uz_ref[...]) + bz_ref[...])
    r = jax.nn.sigmoid(dot(xm, Wr_ref[...]) + dot(red, Ur_ref[...]) + br_ref[...])
    htil = jnp.tanh(dot(xm * r, Wh_ref[...]) + dot(red, Uh_ref[...]) + bh_ref[...])
    h_ref[...] = (1.0 - z) * xb + z * htil


def _phase_c(x, Upair, denpair2, dm, Wz, Uz, bz, Wr, Ur, br, Wh, Uh, bh):
    umap = lambda n: (jnp.where(n < 64, n, n + 8), 0)
    mat = pl.BlockSpec((D, D), lambda n: (0, 0))
    vec = pl.BlockSpec((1, D), lambda n: (0, 0))
    big = pl.BlockSpec((BC, D), lambda n: (n, 0))
    return pl.pallas_call(
        _tc_gru_body,
        grid=(NBC,),
        in_specs=[big, pl.BlockSpec((BC, D), umap),
                  pl.BlockSpec((BC, 1), umap), vec,
                  mat, mat, vec, mat, mat, vec, mat, mat, vec],
        out_specs=big,
        out_shape=jax.ShapeDtypeStruct((N, D), jnp.float32),
    )(x, Upair, denpair2, dm, Wz, Uz, bz, Wr, Ur, br, Wh, Uh, bh)


def kernel(x, edge_index, edge_type, Wrel, attn_l, attn_r, Wz, Uz, bz,
           Wr, Ur, br, Wh, Uh, bh, dropout_mask, step):
    src2 = jnp.pad(edge_index[0], (0, EPAD - E)).reshape(EROWS, 128)
    dst2 = jnp.pad(edge_index[1], (0, EPAD - E)).reshape(EROWS, 128)
    rt2 = jnp.pad(edge_type, (0, EPAD - E)).reshape(EROWS, 128)
    xr_flat, s2_rn, el_n1, pk2 = _phase_a(x, Wrel, attn_l, attn_r,
                                          src2, dst2, rt2)
    s2_flat = s2_rn.reshape(-1)
    el_pad = jnp.pad(el_n1.reshape(-1), (0, NPEL - N))
    pk_pad = pk2.reshape(-1)
    U_pair, den_pair = _sc_phase(xr_flat, s2_flat, el_pad, pk_pad)
    return _phase_c(x, U_pair, den_pair[:, None],
                    dropout_mask.reshape(1, D), Wz, Uz,
                    bz.reshape(1, D), Wr, Ur, br.reshape(1, D), Wh, Uh,
                    bh.reshape(1, D))


# speedup vs baseline: 1.1275x; 1.1275x over previous
import functools
import jax
import jax.numpy as jnp
from jax import lax
from jax.experimental import pallas as pl
from jax.experimental.pallas import tpu as pltpu
from jax.experimental.pallas import tpu_sc as plsc

N = 10000
E = 320000
D = 128
R = 8

NS = 16
NHALF = 5120
EPT = 20480
EPAD = NS * EPT
GB = 64
NSH = 5632
STRIDE = NSH // NS
NPEL = 10256
NB_TC = 10
BN = N // NB_TC
EROWS = 2500


def _tc_pre_body(x_ref, w_ref, al_ref, ar_ref, xr_ref, s2_ref, el_ref):
    xb = x_ref[...]
    w = w_ref[0]
    xr = jnp.dot(xb, w, preferred_element_type=jnp.float32)
    xr_ref[...] = xr
    s2_ref[...] = jnp.dot(xr, ar_ref[...]).reshape(BN, 1)
    el_ref[...] = jnp.dot(xb, al_ref[...]).reshape(BN, 1)


def _phase_a(x, Wrel, attn_l, attn_r):
    return pl.pallas_call(
        _tc_pre_body,
        grid=(R, NB_TC),
        in_specs=[
            pl.BlockSpec((BN, D), lambda r, n: (n, 0)),
            pl.BlockSpec((1, D, D), lambda r, n: (r, 0, 0)),
            pl.BlockSpec((D,), lambda r, n: (0,)),
            pl.BlockSpec((D,), lambda r, n: (0,)),
        ],
        out_specs=[
            pl.BlockSpec((BN, D), lambda r, n: (r * NB_TC + n, 0)),
            pl.BlockSpec((BN, 1), lambda r, n: (r * NB_TC + n, 0)),
            pl.BlockSpec((BN, 1), lambda r, n: (n, 0)),
        ],
        out_shape=[
            jax.ShapeDtypeStruct((R * N, D), jnp.float32),
            jax.ShapeDtypeStruct((R * N, 1), jnp.float32),
            jax.ShapeDtypeStruct((N, 1), jnp.float32),
        ],
    )(x, Wrel, attn_l, attn_r)


def _tc_pack_body(src_ref, dst_ref, rt_ref, pk_ref):
    key = rt_ref[...] * N + src_ref[...]
    pk_ref[...] = key * 16384 + dst_ref[...]


def _phase_pack(src2, dst2, rt2):
    full = pl.BlockSpec((EROWS, 128), lambda: (0, 0))
    return pl.pallas_call(
        _tc_pack_body,
        grid=(),
        in_specs=[full, full, full],
        out_specs=full,
        out_shape=jax.ShapeDtypeStruct((EROWS, 128), jnp.int32),
    )(src2, dst2, rt2)


def _sc_body(xr_h, s2_h, el_h, pk_h, U_h, den_h,
             el_t, pk_t, mpk, rows, srows, kb0, kb1, lb0, lb1, pb0, pb1,
             s2b0, s2b1, zbuf,
             semr0, semr1, sems0, sems1, semu0, semu1, semd0, semd1,
             U_sh, den_sh):
    c = lax.axis_index("c")
    sid = lax.axis_index("s")
    nlo = c * NHALF
    eb = sid * EPT

    zf = jnp.zeros((16,), jnp.float32)
    zi = jnp.zeros((16,), jnp.int32)
    iota16 = lax.iota(jnp.int32, 16)
    m14 = zi + 16383
    s14 = zi + 14
    vH = zi + NHALF
    vE = zi + E
    vT = zi + NHALF
    f02 = zf + 0.2

    pltpu.sync_copy(el_h.at[pl.ds(nlo, NPEL // 2)], el_t)
    pltpu.sync_copy(pk_h.at[pl.ds(eb, EPT)], pk_t)

    @pl.loop(0, 64, unroll=8)
    def _zs(i):
        for j in range(8):
            srows[i, pl.ds(j * 16, 16)] = zf

    @pl.loop(0, STRIDE // 16, unroll=8)
    def _zb(i):
        zbuf[pl.ds(i * 16, 16)] = zf

    pltpu.sync_copy(zbuf, den_sh.at[pl.ds(sid * STRIDE, STRIDE)])
    for k in range(5):
        pltpu.sync_copy(srows.at[pl.ds(0, 64)],
                        U_sh.at[pl.ds(sid * STRIDE + k * 64, 64)])
    pltpu.sync_copy(srows.at[pl.ds(0, 32)],
                    U_sh.at[pl.ds(sid * STRIDE + 320, 32)])
    plsc.subcore_barrier()

    def fbody(i, cnt):
        pk = pk_t[pl.ds(i * 16, 16)]
        ld = (pk & m14) - nlo
        gv = (zi + (eb + i * 16)) + iota16
        msk = (ld >= zi) & (ld < vH) & (gv < vE)
        plsc.store_compressed(mpk.at[pl.ds(cnt, 16)], pk, mask=msk)
        pc = plsc.all_reduce_population_count(msk)[0]
        return cnt + pc

    m = lax.fori_loop(0, EPT // 16, fbody, jnp.int32(0))
    nb = (m + GB - 1) // GB
    mv = jnp.broadcast_to(m, (16,))

    @pl.loop(0, 9)
    def _ztail(i):
        mpk[pl.ds(m + i * 16, 16)] = zi

    def prep_issue(b, kb, rb, semr, sems, s2b):
        boff = b * GB
        for t in range(GB // 16):
            pk = mpk[pl.ds(boff + t * 16, 16)]
            kb[pl.ds(t * 16, 16)] = lax.shift_right_logical(pk, s14)
        pltpu.async_copy(xr_h.at[kb], rows.at[pl.ds(rb, GB)], semr)
        pltpu.async_copy(s2_h.at[kb], s2b, sems)

    def process(b, rb, kb, lb, pb, s2b, semr, sems, semu, semd):
        @pl.when(b >= 2)
        def _():
            pltpu.make_async_copy(srows.at[pl.ds(rb, GB)], U_sh.at[lb],
                                  semu).wait()
            pltpu.make_async_copy(pb.at[pl.ds(0, GB)], den_sh.at[lb],
                                  semd).wait()
        boff = b * GB
        pltpu.make_async_copy(s2_h.at[kb], s2b, sems).wait()
        for t in range(GB // 16):
            pk = mpk[pl.ds(boff + t * 16, 16)]
            ld = (pk & m14) - nlo
            gv = (zi + (boff + t * 16)) + iota16
            ldm = jnp.where(gv < mv, ld, vT)
            lb[pl.ds(t * 16, 16)] = ldm
            eld = plsc.load_gather(el_t, [ldm])
            lg = eld + s2b[pl.ds(t * 16, 16)]
            lr = jnp.where(lg >= zf, lg, lg * f02)
            pb[pl.ds(t * 16, 16)] = jnp.exp(lr)
        pltpu.make_async_copy(xr_h.at[kb], rows.at[pl.ds(rb, GB)],
                              semr).wait()

        @pl.loop(0, GB // 2, unroll=2)
        def _scale(i):
            i2 = 2 * i
            pva = jnp.broadcast_to(pb[pl.ds(i2, 16)][0], (16,))
            pvb = jnp.broadcast_to(pb[pl.ds(i2 + 1, 16)][0], (16,))
            va = [rows[rb + i2, pl.ds(j * 16, 16)] for j in range(8)]
            vb = [rows[rb + i2 + 1, pl.ds(j * 16, 16)] for j in range(8)]
            pa = [pva * v for v in va]
            pb2 = [pvb * v for v in vb]
            for j in range(8):
                srows[rb + i2, pl.ds(j * 16, 16)] = pa[j]
            for j in range(8):
                srows[rb + i2 + 1, pl.ds(j * 16, 16)] = pb2[j]

        pltpu.async_copy(srows.at[pl.ds(rb, GB)], U_sh.at[lb], semu,
                         add=True)
        pltpu.async_copy(pb.at[pl.ds(0, GB)], den_sh.at[lb], semd, add=True)

        @pl.when(b + 2 < nb)
        def _():
            prep_issue(b + 2, kb, rb, semr, sems, s2b)

    @pl.when(nb > 0)
    def _():
        prep_issue(0, kb0, 0, semr0, sems0, s2b0)

    @pl.when(nb > 1)
    def _():
        prep_issue(1, kb1, GB, semr1, sems1, s2b1)

    @pl.loop(0, (nb + 1) // 2)
    def _pairs(u):
        b0 = 2 * u
        b1 = 2 * u + 1
        process(b0, 0, kb0, lb0, pb0, s2b0, semr0, sems0, semu0, semd0)

        @pl.when(b1 < nb)
        def _():
            process(b1, GB, kb1, lb1, pb1, s2b1, semr1, sems1, semu1, semd1)

    @pl.when(nb > 0)
    def _():
        pltpu.make_async_copy(srows.at[pl.ds(0, GB)], U_sh.at[lb0],
                              semu0).wait()
        pltpu.make_async_copy(pb0.at[pl.ds(0, GB)], den_sh.at[lb0],
                              semd0).wait()

    @pl.when(nb > 1)
    def _():
        pltpu.make_async_copy(srows.at[pl.ds(GB, GB)], U_sh.at[lb1],
                              semu1).wait()
        pltpu.make_async_copy(pb1.at[pl.ds(0, GB)], den_sh.at[lb1],
                              semd1).wait()

    plsc.subcore_barrier()

    ob = c * NSH + sid * STRIDE
    pltpu.sync_copy(U_sh.at[pl.ds(sid * STRIDE, STRIDE)],
                    U_h.at[pl.ds(ob, STRIDE)])
    pltpu.sync_copy(den_sh.at[pl.ds(sid * STRIDE, STRIDE)], zbuf)
    pltpu.sync_copy(zbuf, den_h.at[pl.ds(ob, STRIDE)])


_sc_phase = functools.partial(
    pl.kernel,
    out_type=(
        jax.ShapeDtypeStruct((2 * NSH, D), jnp.float32),
        jax.ShapeDtypeStruct((2 * NSH,), jnp.float32),
    ),
    mesh=plsc.VectorSubcoreMesh(core_axis_name="c", subcore_axis_name="s"),
    compiler_params=pltpu.CompilerParams(needs_layout_passes=False,
                                        use_tc_tiling_on_sc=False),
    scratch_types=(
        pltpu.VMEM((NPEL // 2,), jnp.float32),
        pltpu.VMEM((EPT,), jnp.int32),
        pltpu.VMEM((EPT + 160,), jnp.int32),
        pltpu.VMEM((2 * GB, D), jnp.float32),
        pltpu.VMEM((2 * GB, D), jnp.float32),
        pltpu.VMEM((GB,), jnp.int32),
        pltpu.VMEM((GB,), jnp.int32),
        pltpu.VMEM((GB,), jnp.int32),
        pltpu.VMEM((GB,), jnp.int32),
        pltpu.VMEM((GB + 16,), jnp.float32),
        pltpu.VMEM((GB + 16,), jnp.float32),
        pltpu.VMEM((GB,), jnp.float32),
        pltpu.VMEM((GB,), jnp.float32),
        pltpu.VMEM((STRIDE,), jnp.float32),
        pltpu.SemaphoreType.DMA,
        pltpu.SemaphoreType.DMA,
        pltpu.SemaphoreType.DMA,
        pltpu.SemaphoreType.DMA,
        pltpu.SemaphoreType.DMA,
        pltpu.SemaphoreType.DMA,
        pltpu.SemaphoreType.DMA,
        pltpu.SemaphoreType.DMA,
        pltpu.VMEM_SHARED((NSH, D), jnp.float32),
        pltpu.VMEM_SHARED((NSH,), jnp.float32),
    ),
)(_sc_body)


def _tc_gru_body(x_ref, U_ref, den_ref, dm_ref, Wz_ref, Uz_ref, bz_ref,
                 Wr_ref, Ur_ref, br_ref, Wh_ref, Uh_ref, bh_ref, h_ref):
    xb = x_ref[...]
    red = U_ref[...] / (den_ref[...] + 1e-9)
    xm = xb * dm_ref[...]
    dot = lambda a, b: jnp.dot(a, b, preferred_element_type=jnp.float32)
    z = jax.nn.sigmoid(dot(xm, Wz_ref[...]) + dot(red, Uz_ref[...]) + bz_ref[...])
    r = jax.nn.sigmoid(dot(xm, Wr_ref[...]) + dot(red, Ur_ref[...]) + br_ref[...])
    htil = jnp.tanh(dot(xm * r, Wh_ref[...]) + dot(red, Uh_ref[...]) + bh_ref[...])
    h_ref[...] = (1.0 - z) * xb + z * htil


def _phase_c(x, U, den2, dm, Wz, Uz, bz, Wr, Ur, br, Wh, Uh, bh):
    mat = pl.BlockSpec((D, D), lambda n: (0, 0))
    vec = pl.BlockSpec((1, D), lambda n: (0, 0))
    big = pl.BlockSpec((BN, D), lambda n: (n, 0))
    return pl.pallas_call(
        _tc_gru_body,
        grid=(NB_TC,),
        in_specs=[big, big, pl.BlockSpec((BN, 1), lambda n: (n, 0)), vec,
                  mat, mat, vec, mat, mat, vec, mat, mat, vec],
        out_specs=big,
        out_shape=jax.ShapeDtypeStruct((N, D), jnp.float32),
    )(x, U, den2, dm, Wz, Uz, bz, Wr, Ur, br, Wh, Uh, bh)


def kernel(x, edge_index, edge_type, Wrel, attn_l, attn_r, Wz, Uz, bz,
           Wr, Ur, br, Wh, Uh, bh, dropout_mask, step):
    xr_flat, s2_rn, el_n1 = _phase_a(x, Wrel, attn_l, attn_r)
    s2_flat = s2_rn.reshape(-1)
    el_pad = jnp.pad(el_n1.reshape(-1), (0, NPEL - N))
    src2 = edge_index[0].reshape(EROWS, 128)
    dst2 = edge_index[1].reshape(EROWS, 128)
    rt2 = edge_type.reshape(EROWS, 128)
    pk = _phase_pack(src2, dst2, rt2).reshape(-1)
    pk_pad = jnp.pad(pk, (0, EPAD - E))
    U_pair, den_pair = _sc_phase(xr_flat, s2_flat, el_pad, pk_pad)
    U = jnp.concatenate(
        [U_pair[:NHALF], U_pair[NSH:NSH + N - NHALF]], axis=0)
    den = jnp.concatenate(
        [den_pair[:NHALF], den_pair[NSH:NSH + N - NHALF]])
    den2 = den[:, None]
    return _phase_c(x, U, den2, dropout_mask.reshape(1, D), Wz, Uz,
                    bz.reshape(1, D), Wr, Ur, br.reshape(1, D), Wh, Uh,
                    bh.reshape(1, D))
